# baseline (speedup 1.0000x reference)
import functools

import jax
import jax.numpy as jnp
from jax import lax
from jax.experimental import pallas as pl
from jax.experimental.pallas import tpu as pltpu
from jax.experimental.pallas import tpu_sc as plsc

BATCH = 16384
EMBED_DIM = 32
NUM_CORES = 2
NUM_SUBCORES = 16
NUM_WORKERS = NUM_CORES * NUM_SUBCORES
B_PER_W = BATCH // NUM_WORKERS
GATHER_CHUNK = 128
N_GATHER = B_PER_W // GATHER_CHUNK
LANES = 16


def _body(user_ids_hbm, item_ids_hbm, user_table_hbm, item_table_hbm,
          out_hbm, uidx_v, iidx_v, urows_v, irows_v, out_v, sem_u, sem_i):
    wid = lax.axis_index("s") * NUM_CORES + lax.axis_index("c")
    base = wid * B_PER_W

    pltpu.sync_copy(user_ids_hbm.at[pl.ds(base, B_PER_W)], uidx_v)
    pltpu.sync_copy(item_ids_hbm.at[pl.ds(base, B_PER_W)], iidx_v)

    for j in range(N_GATHER):
        sl = pl.ds(j * GATHER_CHUNK, GATHER_CHUNK)
        pltpu.async_copy(user_table_hbm.at[uidx_v.at[sl]], urows_v.at[sl],
                         sem_u)
        pltpu.async_copy(item_table_hbm.at[iidx_v.at[sl]], irows_v.at[sl],
                         sem_i)
    for j in range(N_GATHER):
        sl = pl.ds(j * GATHER_CHUNK, GATHER_CHUNK)
        pltpu.make_async_copy(user_table_hbm.at[uidx_v.at[sl]],
                              urows_v.at[sl], sem_u).wait()
        pltpu.make_async_copy(item_table_hbm.at[iidx_v.at[sl]],
                              irows_v.at[sl], sem_i).wait()

    lane = lax.iota(jnp.int32, LANES)

    def chunk(c, carry):
        accs = jnp.zeros((LANES,), jnp.float32)
        for r in range(LANES):
            b = c * LANES + r
            p = (urows_v[b, pl.ds(0, LANES)] * irows_v[b, pl.ds(0, LANES)]
                 + urows_v[b, pl.ds(LANES, LANES)]
                 * irows_v[b, pl.ds(LANES, LANES)])
            s = jnp.sum(p)
            accs = jnp.where(lane == r, jnp.broadcast_to(s, (LANES,)), accs)
        out_v[pl.ds(c * LANES, LANES)] = accs
        return carry

    lax.fori_loop(0, B_PER_W // LANES, chunk, None)

    pltpu.sync_copy(out_v, out_hbm.at[pl.ds(base, B_PER_W)])


@jax.jit
def kernel(user_ids, item_ids, user_table, item_table):
    mesh = plsc.VectorSubcoreMesh(core_axis_name="c", subcore_axis_name="s")
    k = functools.partial(
        pl.kernel,
        mesh=mesh,
        compiler_params=pltpu.CompilerParams(
            needs_layout_passes=False, use_tc_tiling_on_sc=False),
        out_type=jax.ShapeDtypeStruct((BATCH,), jnp.float32),
        scratch_types=[
            pltpu.VMEM((B_PER_W,), jnp.int32),
            pltpu.VMEM((B_PER_W,), jnp.int32),
            pltpu.VMEM((B_PER_W, EMBED_DIM), jnp.float32),
            pltpu.VMEM((B_PER_W, EMBED_DIM), jnp.float32),
            pltpu.VMEM((B_PER_W,), jnp.float32),
            pltpu.SemaphoreType.DMA,
            pltpu.SemaphoreType.DMA,
        ],
    )(_body)
    return k(user_ids.astype(jnp.int32), item_ids.astype(jnp.int32),
             user_table, item_table)


# speedup vs baseline: 3.8165x; 3.8165x over previous
import functools

import jax
import jax.numpy as jnp
from jax import lax
from jax.experimental import pallas as pl
from jax.experimental.pallas import tpu as pltpu
from jax.experimental.pallas import tpu_sc as plsc

BATCH = 16384
EMBED_DIM = 32
NUM_CORES = 2
NUM_SUBCORES = 16
NUM_WORKERS = NUM_CORES * NUM_SUBCORES
B_PER_W = BATCH // NUM_WORKERS
LANES = 16
GSIZE = 4
N_GROUPS = B_PER_W // GSIZE
IDX_PAD = B_PER_W + LANES


def _body(user_ids_hbm, item_ids_hbm, ut_hbm, it_hbm, out_hbm,
          uidx_v, iidx_v, ubuf, ibuf, acc_v,
          semu0, semu1, semi0, semi1):
    wid = lax.axis_index("s") * NUM_CORES + lax.axis_index("c")
    base = wid * B_PER_W

    pltpu.sync_copy(user_ids_hbm.at[pl.ds(base, B_PER_W)],
                    uidx_v.at[pl.ds(0, B_PER_W)])
    pltpu.sync_copy(item_ids_hbm.at[pl.ds(base, B_PER_W)],
                    iidx_v.at[pl.ds(0, B_PER_W)])

    semus = (semu0, semu1)
    semis = (semi0, semi1)
    lane = lax.iota(jnp.int32, LANES)
    rows0 = lax.iota(jnp.int32, LANES)
    rows1 = rows0 + LANES

    def issue(g, slot):
        uvec = uidx_v[pl.ds(g * GSIZE, LANES)]
        ivec = iidx_v[pl.ds(g * GSIZE, LANES)]
        for k in range(GSIZE):
            uoff = pl.multiple_of((uvec[k] >> 7) << 7, 128)
            ioff = pl.multiple_of((ivec[k] >> 7) << 7, 128)
            pltpu.async_copy(ut_hbm.at[:, pl.ds(uoff, 128)],
                             ubuf.at[slot, k], semus[slot])
            pltpu.async_copy(it_hbm.at[:, pl.ds(ioff, 128)],
                             ibuf.at[slot, k], semis[slot])

    def drain(slot):
        for k in range(GSIZE):
            pltpu.make_async_copy(ut_hbm.at[:, pl.ds(0, 128)],
                                  ubuf.at[slot, k], semus[slot]).wait()
            pltpu.make_async_copy(it_hbm.at[:, pl.ds(0, 128)],
                                  ibuf.at[slot, k], semis[slot]).wait()

    def compute(g, slot):
        uvec = uidx_v[pl.ds(g * GSIZE, LANES)]
        ivec = iidx_v[pl.ds(g * GSIZE, LANES)]
        ulane = uvec & 127
        ilane = ivec & 127
        sl = pl.ds((g >> 2) * LANES, LANES)
        acc = acc_v[sl]
        lbase = (g & 3) * GSIZE
        for k in range(GSIZE):
            cu = jnp.full((LANES,), ulane[k], jnp.int32)
            ci = jnp.full((LANES,), ilane[k], jnp.int32)
            u0 = plsc.load_gather(ubuf.at[slot, k], [rows0, cu])
            u1 = plsc.load_gather(ubuf.at[slot, k], [rows1, cu])
            v0 = plsc.load_gather(ibuf.at[slot, k], [rows0, ci])
            v1 = plsc.load_gather(ibuf.at[slot, k], [rows1, ci])
            s = jnp.sum(u0 * v0 + u1 * v1)
            acc = jnp.where(lane == lbase + k,
                            jnp.broadcast_to(s, (LANES,)), acc)
        acc_v[sl] = acc

    issue(0, 0)
    issue(1, 1)

    def superstep(t, carry):
        for j in range(2):
            g = 2 * t + j
            drain(j)
            compute(g, j)

            @pl.when(t < N_GROUPS // 2 - 1)
            def _(j=j, g=g):
                issue(g + 2, j)

        return carry

    lax.fori_loop(0, N_GROUPS // 2, superstep, None)

    pltpu.sync_copy(acc_v, out_hbm.at[pl.ds(base, B_PER_W)])


@jax.jit
def kernel(user_ids, item_ids, user_table, item_table):
    mesh = plsc.VectorSubcoreMesh(core_axis_name="c", subcore_axis_name="s")
    k = functools.partial(
        pl.kernel,
        mesh=mesh,
        compiler_params=pltpu.CompilerParams(needs_layout_passes=False),
        out_type=jax.ShapeDtypeStruct((BATCH,), jnp.float32),
        scratch_types=[
            pltpu.VMEM((IDX_PAD,), jnp.int32),
            pltpu.VMEM((IDX_PAD,), jnp.int32),
            pltpu.VMEM((2, GSIZE, EMBED_DIM, 128), jnp.float32),
            pltpu.VMEM((2, GSIZE, EMBED_DIM, 128), jnp.float32),
            pltpu.VMEM((B_PER_W,), jnp.float32),
            pltpu.SemaphoreType.DMA,
            pltpu.SemaphoreType.DMA,
            pltpu.SemaphoreType.DMA,
            pltpu.SemaphoreType.DMA,
        ],
    )(_body)
    return k(user_ids.astype(jnp.int32), item_ids.astype(jnp.int32),
             user_table.T, item_table.T)


# speedup vs baseline: 4.2077x; 1.1025x over previous
import functools

import jax
import jax.numpy as jnp
from jax import lax
from jax.experimental import pallas as pl
from jax.experimental.pallas import tpu as pltpu
from jax.experimental.pallas import tpu_sc as plsc

BATCH = 16384
EMBED_DIM = 32
NUM_CORES = 2
NUM_SUBCORES = 16
NUM_WORKERS = NUM_CORES * NUM_SUBCORES
B_PER_W = BATCH // NUM_WORKERS
LANES = 16
GSIZE = 4
NSLOT = 3
N_GROUPS = B_PER_W // GSIZE
IDX_PAD = B_PER_W + LANES


def _body(user_ids_hbm, item_ids_hbm, ut_hbm, it_hbm, out_hbm,
          uidx_v, iidx_v, ubuf, ibuf, acc_v,
          semu0, semu1, semu2, semi0, semi1, semi2):
    wid = lax.axis_index("s") * NUM_CORES + lax.axis_index("c")
    base = wid * B_PER_W

    pltpu.sync_copy(user_ids_hbm.at[pl.ds(base, B_PER_W)],
                    uidx_v.at[pl.ds(0, B_PER_W)])
    pltpu.sync_copy(item_ids_hbm.at[pl.ds(base, B_PER_W)],
                    iidx_v.at[pl.ds(0, B_PER_W)])

    semus = (semu0, semu1, semu2)
    semis = (semi0, semi1, semi2)
    lane = lax.iota(jnp.int32, LANES)
    rows0 = lax.iota(jnp.int32, LANES)
    rows1 = rows0 + LANES

    def issue(g, slot):
        uvec = uidx_v[pl.ds(g * GSIZE, LANES)]
        ivec = iidx_v[pl.ds(g * GSIZE, LANES)]
        for k in range(GSIZE):
            uoff = pl.multiple_of((uvec[k] >> 7) << 7, 128)
            ioff = pl.multiple_of((ivec[k] >> 7) << 7, 128)
            pltpu.async_copy(ut_hbm.at[:, pl.ds(uoff, 128)],
                             ubuf.at[slot, k], semus[slot])
            pltpu.async_copy(it_hbm.at[:, pl.ds(ioff, 128)],
                             ibuf.at[slot, k], semis[slot])

    def drain(slot):
        for k in range(GSIZE):
            pltpu.make_async_copy(ut_hbm.at[:, pl.ds(0, 128)],
                                  ubuf.at[slot, k], semus[slot]).wait()
            pltpu.make_async_copy(it_hbm.at[:, pl.ds(0, 128)],
                                  ibuf.at[slot, k], semis[slot]).wait()

    def compute(g, slot):
        uvec = uidx_v[pl.ds(g * GSIZE, LANES)]
        ivec = iidx_v[pl.ds(g * GSIZE, LANES)]
        ulane = uvec & 127
        ilane = ivec & 127
        sl = pl.ds((g >> 2) * LANES, LANES)
        acc = acc_v[sl]
        lbase = (g & 3) * GSIZE
        for k in range(GSIZE):
            cu = jnp.full((LANES,), ulane[k], jnp.int32)
            ci = jnp.full((LANES,), ilane[k], jnp.int32)
            u0 = plsc.load_gather(ubuf.at[slot, k], [rows0, cu])
            u1 = plsc.load_gather(ubuf.at[slot, k], [rows1, cu])
            v0 = plsc.load_gather(ibuf.at[slot, k], [rows0, ci])
            v1 = plsc.load_gather(ibuf.at[slot, k], [rows1, ci])
            s = jnp.sum(u0 * v0 + u1 * v1)
            acc = jnp.where(lane == lbase + k,
                            jnp.broadcast_to(s, (LANES,)), acc)
        acc_v[sl] = acc

    issue(0, 0)
    issue(1, 1)
    issue(2, 2)

    def superstep(t, carry):
        for j in range(NSLOT):
            g = NSLOT * t + j
            drain(j)
            compute(g, j)

            @pl.when(g < N_GROUPS - NSLOT)
            def _(j=j, g=g):
                issue(g + NSLOT, j)

        return carry

    n_steps = (N_GROUPS - 2) // NSLOT
    lax.fori_loop(0, n_steps, superstep, None)
    for g in range(NSLOT * n_steps, N_GROUPS):
        drain(g % NSLOT)
        compute(jnp.int32(g), g % NSLOT)

    pltpu.sync_copy(acc_v, out_hbm.at[pl.ds(base, B_PER_W)])


@jax.jit
def kernel(user_ids, item_ids, user_table, item_table):
    mesh = plsc.VectorSubcoreMesh(core_axis_name="c", subcore_axis_name="s")
    k = functools.partial(
        pl.kernel,
        mesh=mesh,
        compiler_params=pltpu.CompilerParams(needs_layout_passes=False),
        out_type=jax.ShapeDtypeStruct((BATCH,), jnp.float32),
        scratch_types=[
            pltpu.VMEM((IDX_PAD,), jnp.int32),
            pltpu.VMEM((IDX_PAD,), jnp.int32),
            pltpu.VMEM((NSLOT, GSIZE, EMBED_DIM, 128), jnp.float32),
            pltpu.VMEM((NSLOT, GSIZE, EMBED_DIM, 128), jnp.float32),
            pltpu.VMEM((B_PER_W,), jnp.float32),
            pltpu.SemaphoreType.DMA,
            pltpu.SemaphoreType.DMA,
            pltpu.SemaphoreType.DMA,
            pltpu.SemaphoreType.DMA,
            pltpu.SemaphoreType.DMA,
            pltpu.SemaphoreType.DMA,
        ],
    )(_body)
    return k(user_ids.astype(jnp.int32), item_ids.astype(jnp.int32),
             user_table.T, item_table.T)


# speedup vs baseline: 4.7230x; 1.1225x over previous
import functools

import jax
import jax.numpy as jnp
from jax import lax
from jax.experimental import pallas as pl
from jax.experimental.pallas import tpu as pltpu
from jax.experimental.pallas import tpu_sc as plsc

BATCH = 16384
EMBED_DIM = 32
NUM_CORES = 2
NUM_SUBCORES = 16
NUM_WORKERS = NUM_CORES * NUM_SUBCORES
B_PER_W = BATCH // NUM_WORKERS
LANES = 16
GSIZE = 4
NSLOT = 3
N_GROUPS = B_PER_W // GSIZE
IDX_PAD = B_PER_W + LANES


def _gather_body(su_hbm, fu_hbm, bu_hbm, pu_hbm,
                 si_hbm, fi_hbm, bi_hbm, pi_hbm,
                 ut_hbm, it_hbm, uembf_hbm, iembf_hbm,
                 suv, fuv, buv, puv, siv, fiv, biv, piv,
                 ubuf, ibuf, ustage, istage,
                 semu0, semu1, semu2, semi0, semi1, semi2,
                 osemu0, osemu1, osemu2, osemi0, osemi1, osemi2):
    wid = lax.axis_index("s") * NUM_CORES + lax.axis_index("c")
    base = wid * B_PER_W

    for src, dst in ((su_hbm, suv), (fu_hbm, fuv), (bu_hbm, buv),
                     (pu_hbm, puv), (si_hbm, siv), (fi_hbm, fiv),
                     (bi_hbm, biv), (pi_hbm, piv)):
        pltpu.sync_copy(src.at[pl.ds(base, B_PER_W)],
                        dst.at[pl.ds(0, B_PER_W)])

    semus = (semu0, semu1, semu2)
    semis = (semi0, semi1, semi2)
    osemus = (osemu0, osemu1, osemu2)
    osemis = (osemi0, osemi1, osemi2)
    rows0 = lax.iota(jnp.int32, LANES)
    rows1 = rows0 + LANES

    def issue(g, slot):
        for sv, fv, buf, sems in ((suv, fuv, ubuf, semus),
                                  (siv, fiv, ibuf, semis)):
            svec = sv[pl.ds(g * GSIZE, LANES)]
            fvec = fv[pl.ds(g * GSIZE, LANES)]
            for k in range(GSIZE):
                @pl.when(fvec[k] != 0)
                def _(svec=svec, k=k, buf=buf, sems=sems):
                    off = pl.multiple_of((svec[k] >> 7) << 7, 128)
                    tab = ut_hbm if buf is ubuf else it_hbm
                    pltpu.async_copy(tab.at[:, pl.ds(off, 128)],
                                     buf.at[slot, k], sems[slot])

    def drain(g, slot):
        for fv, buf, sems in ((fuv, ubuf, semus), (fiv, ibuf, semis)):
            fvec = fv[pl.ds(g * GSIZE, LANES)]
            for k in range(GSIZE):
                @pl.when(fvec[k] != 0)
                def _(k=k, buf=buf, sems=sems):
                    tab = ut_hbm if buf is ubuf else it_hbm
                    pltpu.make_async_copy(tab.at[:, pl.ds(0, 128)],
                                          buf.at[slot, k],
                                          sems[slot]).wait()

    def drain_out(g, slot):
        @pl.when(g >= NSLOT)
        def _():
            for stage, emb, osems in ((ustage, uembf_hbm, osemus),
                                      (istage, iembf_hbm, osemis)):
                for k in range(GSIZE):
                    pltpu.make_async_copy(stage.at[slot, k],
                                          emb.at[pl.ds(0, EMBED_DIM)],
                                          osems[slot]).wait()

    def compute(g, slot):
        for sv, bv, pv, buf, stage, emb, osems in (
                (suv, buv, puv, ubuf, ustage, uembf_hbm, osemus),
                (siv, biv, piv, ibuf, istage, iembf_hbm, osemis)):
            svec = sv[pl.ds(g * GSIZE, LANES)]
            bvec = bv[pl.ds(g * GSIZE, LANES)]
            pvec = pv[pl.ds(g * GSIZE, LANES)]
            cvec = svec & 127
            for k in range(GSIZE):
                ref = buf.at[slot, bvec[k]]
                c = jnp.full((LANES,), cvec[k], jnp.int32)
                e0 = plsc.load_gather(ref, [rows0, c])
                e1 = plsc.load_gather(ref, [rows1, c])
                stage[slot, k, pl.ds(0, LANES)] = e0
                stage[slot, k, pl.ds(LANES, LANES)] = e1
                pltpu.async_copy(stage.at[slot, k],
                                 emb.at[pl.ds(pvec[k] * EMBED_DIM,
                                              EMBED_DIM)],
                                 osems[slot])

    issue(jnp.int32(0), 0)
    issue(jnp.int32(1), 1)
    issue(jnp.int32(2), 2)

    def superstep(t, carry):
        for j in range(NSLOT):
            g = NSLOT * t + j
            drain(g, j)
            drain_out(g, j)
            compute(g, j)

            @pl.when(g < N_GROUPS - NSLOT)
            def _(j=j, g=g):
                issue(g + NSLOT, j)

        return carry

    n_steps = (N_GROUPS - 2) // NSLOT
    lax.fori_loop(0, n_steps, superstep, None)
    for g in range(NSLOT * n_steps, N_GROUPS):
        drain(jnp.int32(g), g % NSLOT)
        drain_out(jnp.int32(g), g % NSLOT)
        compute(jnp.int32(g), g % NSLOT)
    for slot in range(NSLOT):
        for stage, emb, osems in ((ustage, uembf_hbm, osemus),
                                  (istage, iembf_hbm, osemis)):
            for k in range(GSIZE):
                pltpu.make_async_copy(stage.at[slot, k],
                                      emb.at[pl.ds(0, EMBED_DIM)],
                                      osems[slot]).wait()


def _dot_body(uembf_hbm, iembf_hbm, out_hbm, uv, iv, acc_v):
    wid = lax.axis_index("s") * NUM_CORES + lax.axis_index("c")
    base = wid * B_PER_W

    pltpu.sync_copy(uembf_hbm.at[pl.ds(base * EMBED_DIM,
                                       B_PER_W * EMBED_DIM)], uv)
    pltpu.sync_copy(iembf_hbm.at[pl.ds(base * EMBED_DIM,
                                       B_PER_W * EMBED_DIM)], iv)

    lanes = lax.iota(jnp.int32, LANES)

    def chunk(c, carry):
        rowbase = (c * LANES + lanes) * EMBED_DIM
        acc = jnp.zeros((LANES,), jnp.float32)
        for d in range(EMBED_DIM):
            u = plsc.load_gather(uv, [rowbase + d])
            v = plsc.load_gather(iv, [rowbase + d])
            acc = acc + u * v
        acc_v[pl.ds(c * LANES, LANES)] = acc
        return carry

    lax.fori_loop(0, B_PER_W // LANES, chunk, None)
    pltpu.sync_copy(acc_v, out_hbm.at[pl.ds(base, B_PER_W)])


def _prep(ids):
    iot = lax.iota(jnp.int32, BATCH)
    s, p = lax.sort_key_val(ids, iot)
    q = s >> 7
    newcol = jnp.concatenate(
        [jnp.ones((1,), jnp.int32), (q[1:] != q[:-1]).astype(jnp.int32)])
    flag = newcol | ((iot & (GSIZE - 1)) == 0).astype(jnp.int32)
    pos = iot & (GSIZE - 1)
    bidx = lax.cummax(jnp.where(flag != 0, pos, 0).reshape(-1, GSIZE),
                      axis=1).reshape(-1)
    return s, flag, bidx.astype(jnp.int32), p


@jax.jit
def kernel(user_ids, item_ids, user_table, item_table):
    su, fu, bu, pu = _prep(user_ids.astype(jnp.int32))
    si, fi, bi, pi = _prep(item_ids.astype(jnp.int32))

    mesh = plsc.VectorSubcoreMesh(core_axis_name="c", subcore_axis_name="s")
    params = pltpu.CompilerParams(needs_layout_passes=False)

    gather_k = functools.partial(
        pl.kernel,
        mesh=mesh,
        compiler_params=params,
        out_type=(
            jax.ShapeDtypeStruct((BATCH * EMBED_DIM,), jnp.float32),
            jax.ShapeDtypeStruct((BATCH * EMBED_DIM,), jnp.float32),
        ),
        scratch_types=(
            [pltpu.VMEM((IDX_PAD,), jnp.int32) for _ in range(8)]
            + [
                pltpu.VMEM((NSLOT, GSIZE, EMBED_DIM, 128), jnp.float32),
                pltpu.VMEM((NSLOT, GSIZE, EMBED_DIM, 128), jnp.float32),
                pltpu.VMEM((NSLOT, GSIZE, EMBED_DIM), jnp.float32),
                pltpu.VMEM((NSLOT, GSIZE, EMBED_DIM), jnp.float32),
            ]
            + [pltpu.SemaphoreType.DMA for _ in range(12)]
        ),
    )(_gather_body)
    uembf, iembf = gather_k(su, fu, bu, pu, si, fi, bi, pi,
                            user_table.T, item_table.T)

    dot_k = functools.partial(
        pl.kernel,
        mesh=mesh,
        compiler_params=params,
        out_type=jax.ShapeDtypeStruct((BATCH,), jnp.float32),
        scratch_types=[
            pltpu.VMEM((B_PER_W * EMBED_DIM,), jnp.float32),
            pltpu.VMEM((B_PER_W * EMBED_DIM,), jnp.float32),
            pltpu.VMEM((B_PER_W,), jnp.float32),
        ],
    )(_dot_body)
    return dot_k(uembf, iembf)


# speedup vs baseline: 5.1189x; 1.0838x over previous
import functools

import jax
import jax.numpy as jnp
from jax import lax
from jax.experimental import pallas as pl
from jax.experimental.pallas import tpu as pltpu
from jax.experimental.pallas import tpu_sc as plsc

BATCH = 16384
EMBED_DIM = 32
NUM_CORES = 2
NUM_SUBCORES = 16
NUM_WORKERS = NUM_CORES * NUM_SUBCORES
B_PER_W = BATCH // NUM_WORKERS
LANES = 16
GSIZE = 4
NSLOT = 3
N_GROUPS = B_PER_W // GSIZE
IDX_PAD = B_PER_W + LANES


def _gather_body(su_hbm, fu_hbm, bu_hbm, pu_hbm,
                 si_hbm, fi_hbm, bi_hbm, pi_hbm,
                 ut_hbm, it_hbm, uembf_hbm, iembf_hbm,
                 suv, fuv, buv, puv, siv, fiv, biv, piv,
                 ubuf, ibuf, ustage, istage,
                 semu0, semu1, semu2, semi0, semi1, semi2,
                 osemu0, osemu1, osemu2, osemi0, osemi1, osemi2):
    wid = lax.axis_index("s") * NUM_CORES + lax.axis_index("c")
    base = wid * B_PER_W

    for src, dst in ((su_hbm, suv), (fu_hbm, fuv), (bu_hbm, buv),
                     (pu_hbm, puv), (si_hbm, siv), (fi_hbm, fiv),
                     (bi_hbm, biv), (pi_hbm, piv)):
        pltpu.sync_copy(src.at[pl.ds(base, B_PER_W)],
                        dst.at[pl.ds(0, B_PER_W)])

    semus = (semu0, semu1, semu2)
    semis = (semi0, semi1, semi2)
    osemus = (osemu0, osemu1, osemu2)
    osemis = (osemi0, osemi1, osemi2)
    rows0 = lax.iota(jnp.int32, LANES)
    rows1 = rows0 + LANES

    def issue(g, slot):
        for sv, fv, buf, sems in ((suv, fuv, ubuf, semus),
                                  (siv, fiv, ibuf, semis)):
            svec = sv[pl.ds(g * GSIZE, LANES)]
            fvec = fv[pl.ds(g * GSIZE, LANES)]
            for k in range(GSIZE):
                @pl.when(fvec[k] != 0)
                def _(svec=svec, k=k, buf=buf, sems=sems):
                    off = pl.multiple_of((svec[k] >> 7) << 7, 128)
                    tab = ut_hbm if buf is ubuf else it_hbm
                    pltpu.async_copy(tab.at[:, pl.ds(off, 128)],
                                     buf.at[slot, k], sems[slot])

    def drain(g, slot):
        for fv, buf, sems in ((fuv, ubuf, semus), (fiv, ibuf, semis)):
            fvec = fv[pl.ds(g * GSIZE, LANES)]
            for k in range(GSIZE):
                @pl.when(fvec[k] != 0)
                def _(k=k, buf=buf, sems=sems):
                    tab = ut_hbm if buf is ubuf else it_hbm
                    pltpu.make_async_copy(tab.at[:, pl.ds(0, 128)],
                                          buf.at[slot, k],
                                          sems[slot]).wait()

    def drain_out(g, slot):
        @pl.when(g >= NSLOT)
        def _():
            for stage, emb, osems in ((ustage, uembf_hbm, osemus),
                                      (istage, iembf_hbm, osemis)):
                for k in range(GSIZE):
                    pltpu.make_async_copy(stage.at[slot, k],
                                          emb.at[pl.ds(0, EMBED_DIM)],
                                          osems[slot]).wait()

    def compute(g, slot):
        for sv, bv, pv, buf, stage, emb, osems in (
                (suv, buv, puv, ubuf, ustage, uembf_hbm, osemus),
                (siv, biv, piv, ibuf, istage, iembf_hbm, osemis)):
            svec = sv[pl.ds(g * GSIZE, LANES)]
            bvec = bv[pl.ds(g * GSIZE, LANES)]
            pvec = pv[pl.ds(g * GSIZE, LANES)]
            cvec = svec & 127
            for k in range(GSIZE):
                ref = buf.at[slot, bvec[k]]
                c = jnp.full((LANES,), cvec[k], jnp.int32)
                e0 = plsc.load_gather(ref, [rows0, c])
                e1 = plsc.load_gather(ref, [rows1, c])
                stage[slot, k, pl.ds(0, LANES)] = e0
                stage[slot, k, pl.ds(LANES, LANES)] = e1
                pltpu.async_copy(stage.at[slot, k],
                                 emb.at[pl.ds(pvec[k] * EMBED_DIM,
                                              EMBED_DIM)],
                                 osems[slot])

    issue(jnp.int32(0), 0)
    issue(jnp.int32(1), 1)
    issue(jnp.int32(2), 2)

    def superstep(t, carry):
        for j in range(NSLOT):
            g = NSLOT * t + j
            drain(g, j)
            drain_out(g, j)
            compute(g, j)

            @pl.when(g < N_GROUPS - NSLOT)
            def _(j=j, g=g):
                issue(g + NSLOT, j)

        return carry

    n_steps = (N_GROUPS - 2) // NSLOT
    lax.fori_loop(0, n_steps, superstep, None)
    for g in range(NSLOT * n_steps, N_GROUPS):
        drain(jnp.int32(g), g % NSLOT)
        drain_out(jnp.int32(g), g % NSLOT)
        compute(jnp.int32(g), g % NSLOT)
    for slot in range(NSLOT):
        for stage, emb, osems in ((ustage, uembf_hbm, osemus),
                                  (istage, iembf_hbm, osemis)):
            for k in range(GSIZE):
                pltpu.make_async_copy(stage.at[slot, k],
                                      emb.at[pl.ds(0, EMBED_DIM)],
                                      osems[slot]).wait()


def _dot_body(uembf_hbm, iembf_hbm, out_hbm, uv, iv, acc_v):
    wid = lax.axis_index("s") * NUM_CORES + lax.axis_index("c")
    base = wid * B_PER_W

    pltpu.sync_copy(uembf_hbm.at[pl.ds(base * EMBED_DIM,
                                       B_PER_W * EMBED_DIM)], uv)
    pltpu.sync_copy(iembf_hbm.at[pl.ds(base * EMBED_DIM,
                                       B_PER_W * EMBED_DIM)], iv)

    lane = lax.iota(jnp.int32, LANES)

    def chunk(c, carry):
        accs = jnp.zeros((LANES,), jnp.float32)
        for r in range(LANES):
            rb = (c * LANES + r) * EMBED_DIM
            p = (uv[pl.ds(rb, LANES)] * iv[pl.ds(rb, LANES)]
                 + uv[pl.ds(rb + LANES, LANES)]
                 * iv[pl.ds(rb + LANES, LANES)])
            s = jnp.sum(p)
            accs = jnp.where(lane == r, jnp.broadcast_to(s, (LANES,)), accs)
        acc_v[pl.ds(c * LANES, LANES)] = accs
        return carry

    lax.fori_loop(0, B_PER_W // LANES, chunk, None)
    pltpu.sync_copy(acc_v, out_hbm.at[pl.ds(base, B_PER_W)])


def _prep(ids):
    iot = lax.iota(jnp.int32, BATCH)
    s, p = lax.sort_key_val(ids, iot)
    q = s >> 7
    newcol = jnp.concatenate(
        [jnp.ones((1,), jnp.int32), (q[1:] != q[:-1]).astype(jnp.int32)])
    flag = newcol | ((iot & (GSIZE - 1)) == 0).astype(jnp.int32)
    pos = iot & (GSIZE - 1)
    bidx = lax.cummax(jnp.where(flag != 0, pos, 0).reshape(-1, GSIZE),
                      axis=1).reshape(-1)
    return s, flag, bidx.astype(jnp.int32), p


@jax.jit
def kernel(user_ids, item_ids, user_table, item_table):
    su, fu, bu, pu = _prep(user_ids.astype(jnp.int32))
    si, fi, bi, pi = _prep(item_ids.astype(jnp.int32))

    mesh = plsc.VectorSubcoreMesh(core_axis_name="c", subcore_axis_name="s")
    params = pltpu.CompilerParams(needs_layout_passes=False)

    gather_k = functools.partial(
        pl.kernel,
        mesh=mesh,
        compiler_params=params,
        out_type=(
            jax.ShapeDtypeStruct((BATCH * EMBED_DIM,), jnp.float32),
            jax.ShapeDtypeStruct((BATCH * EMBED_DIM,), jnp.float32),
        ),
        scratch_types=(
            [pltpu.VMEM((IDX_PAD,), jnp.int32) for _ in range(8)]
            + [
                pltpu.VMEM((NSLOT, GSIZE, EMBED_DIM, 128), jnp.float32),
                pltpu.VMEM((NSLOT, GSIZE, EMBED_DIM, 128), jnp.float32),
                pltpu.VMEM((NSLOT, GSIZE, EMBED_DIM), jnp.float32),
                pltpu.VMEM((NSLOT, GSIZE, EMBED_DIM), jnp.float32),
            ]
            + [pltpu.SemaphoreType.DMA for _ in range(12)]
        ),
    )(_gather_body)
    uembf, iembf = gather_k(su, fu, bu, pu, si, fi, bi, pi,
                            user_table.T, item_table.T)

    dot_k = functools.partial(
        pl.kernel,
        mesh=mesh,
        compiler_params=params,
        out_type=jax.ShapeDtypeStruct((BATCH,), jnp.float32),
        scratch_types=[
            pltpu.VMEM((B_PER_W * EMBED_DIM,), jnp.float32),
            pltpu.VMEM((B_PER_W * EMBED_DIM,), jnp.float32),
            pltpu.VMEM((B_PER_W,), jnp.float32),
        ],
    )(_dot_body)
    return dot_k(uembf, iembf)
